# Initial kernel scaffold; baseline (speedup 1.0000x reference)
import jax
import jax.numpy as jnp
from jax.experimental import pallas as pl


def kernel(species, coordinates, conv_tensor):
    raise NotImplementedError("write your pallas kernel here")


# speedup vs baseline: 237.9683x; 237.9683x over previous
import functools

import jax
import jax.numpy as jnp
from jax import lax
from jax.experimental import pallas as pl
from jax.experimental.pallas import tpu as pltpu
from jax.experimental.pallas import tpu_sc as plsc

_L = 16
_NC = 2
_NS = 16
_NW = _NC * _NS
_N = 8192 * 512
_PER_W = _N // _NW
_CHUNK = 65536
_NCHUNKS = _PER_W // _CHUNK


def _sc_table_gather(species_flat, conv):
    mesh = plsc.VectorSubcoreMesh(core_axis_name="c", subcore_axis_name="s")

    @functools.partial(
        pl.kernel,
        mesh=mesh,
        out_type=jax.ShapeDtypeStruct((_N,), jnp.int32),
        compiler_params=pltpu.CompilerParams(needs_layout_passes=False),
        scratch_types=[
            pltpu.VMEM((128,), jnp.int32),
            pltpu.VMEM((_CHUNK,), jnp.int32),
        ],
    )
    def k(species_hbm, conv_hbm, out_hbm, conv_v, buf_v):
        wid = lax.axis_index("s") * _NC + lax.axis_index("c")
        base = wid * _PER_W
        pltpu.sync_copy(conv_hbm, conv_v)

        def chunk_body(ci, _):
            off = base + ci * _CHUNK
            pltpu.sync_copy(species_hbm.at[pl.ds(off, _CHUNK)], buf_v)

            def body(i, _):
                idx = buf_v[pl.ds(i * _L, _L)]
                buf_v[pl.ds(i * _L, _L)] = plsc.load_gather(conv_v, [idx])
                return 0

            lax.fori_loop(0, _CHUNK // _L, body, 0)
            pltpu.sync_copy(buf_v, out_hbm.at[pl.ds(off, _CHUNK)])
            return 0

        lax.fori_loop(0, _NCHUNKS, chunk_body, 0)

    return k(species_flat, conv)


def kernel(species, coordinates, conv_tensor):
    species_flat = species.astype(jnp.int32).reshape(-1)
    conv = jnp.pad(conv_tensor.astype(jnp.int32), (0, 128 - conv_tensor.shape[0]))
    out = _sc_table_gather(species_flat, conv)
    return out.reshape(species.shape).astype(conv_tensor.dtype), coordinates


# speedup vs baseline: 269.7676x; 1.1336x over previous
import functools

import jax
import jax.numpy as jnp
from jax import lax
from jax.experimental import pallas as pl
from jax.experimental.pallas import tpu as pltpu
from jax.experimental.pallas import tpu_sc as plsc

_L = 16
_NC = 2
_NS = 16
_NW = _NC * _NS
_N = 8192 * 512
_PER_W = _N // _NW
_CHUNK = 16384
_NCHUNKS = _PER_W // _CHUNK


def _sc_table_gather(species_flat, conv):
    mesh = plsc.VectorSubcoreMesh(core_axis_name="c", subcore_axis_name="s")

    @functools.partial(
        pl.kernel,
        mesh=mesh,
        out_type=jax.ShapeDtypeStruct((_N,), jnp.int32),
        compiler_params=pltpu.CompilerParams(needs_layout_passes=False),
        scratch_types=[
            pltpu.VMEM((128,), jnp.int32),
            pltpu.VMEM((_CHUNK,), jnp.int32),
            pltpu.VMEM((_CHUNK,), jnp.int32),
            pltpu.VMEM((_CHUNK,), jnp.int32),
            pltpu.VMEM((_CHUNK,), jnp.int32),
            pltpu.SemaphoreType.DMA,
            pltpu.SemaphoreType.DMA,
            pltpu.SemaphoreType.DMA,
            pltpu.SemaphoreType.DMA,
        ],
    )
    def k(species_hbm, conv_hbm, out_hbm, conv_v,
          in0, in1, out0, out1, si0, si1, so0, so1):
        wid = lax.axis_index("s") * _NC + lax.axis_index("c")
        base = wid * _PER_W
        pltpu.sync_copy(conv_hbm, conv_v)

        in_bufs, out_bufs = (in0, in1), (out0, out1)
        in_sems, out_sems = (si0, si1), (so0, so1)

        def start_in(ci):
            return pltpu.async_copy(
                species_hbm.at[pl.ds(base + ci * _CHUNK, _CHUNK)],
                in_bufs[ci % 2], in_sems[ci % 2])

        def start_out(ci):
            return pltpu.async_copy(
                out_bufs[ci % 2],
                out_hbm.at[pl.ds(base + ci * _CHUNK, _CHUNK)],
                out_sems[ci % 2])

        def translate(src, dst):
            def body(i, _):
                idx = src[pl.ds(i * _L, _L)]
                dst[pl.ds(i * _L, _L)] = plsc.load_gather(conv_v, [idx])
                return 0
            lax.fori_loop(0, _CHUNK // _L, body, 0, unroll=8)

        pending_in = {0: start_in(0)}
        pending_out = {}
        for ci in range(_NCHUNKS):
            b = ci % 2
            if ci + 1 < _NCHUNKS:
                pending_in[ci + 1] = start_in(ci + 1)
            pending_in.pop(ci).wait()
            if ci - 2 in pending_out:
                pending_out.pop(ci - 2).wait()
            translate(in_bufs[b], out_bufs[b])
            pending_out[ci] = start_out(ci)
        for ci in sorted(pending_out):
            pending_out.pop(ci).wait()

    return k(species_flat, conv)


def kernel(species, coordinates, conv_tensor):
    species_flat = species.astype(jnp.int32).reshape(-1)
    conv = jnp.pad(conv_tensor.astype(jnp.int32), (0, 128 - conv_tensor.shape[0]))
    out = _sc_table_gather(species_flat, conv)
    return out.reshape(species.shape).astype(conv_tensor.dtype), coordinates


# speedup vs baseline: 412.1132x; 1.5277x over previous
import functools

import jax
import jax.numpy as jnp
from jax import lax
from jax.experimental import pallas as pl
from jax.experimental.pallas import tpu as pltpu
from jax.experimental.pallas import tpu_sc as plsc

_L = 16
_NC = 2
_NS = 16
_NW = _NC * _NS
_N = 8192 * 512
_PER_W = _N // _NW
_CHUNK = 16384
_NCHUNKS = _PER_W // _CHUNK


def _sc_table_gather(species_flat, conv):
    mesh = plsc.VectorSubcoreMesh(core_axis_name="c", subcore_axis_name="s")

    @functools.partial(
        pl.kernel,
        mesh=mesh,
        out_type=jax.ShapeDtypeStruct((_N,), jnp.int32),
        compiler_params=pltpu.CompilerParams(needs_layout_passes=False),
        scratch_types=[
            pltpu.VMEM((128,), jnp.int32),
            pltpu.VMEM((_CHUNK,), jnp.int32),
            pltpu.VMEM((_CHUNK,), jnp.int32),
            pltpu.VMEM((_CHUNK,), jnp.int32),
            pltpu.VMEM((_CHUNK,), jnp.int32),
            pltpu.SemaphoreType.DMA,
            pltpu.SemaphoreType.DMA,
            pltpu.SemaphoreType.DMA,
            pltpu.SemaphoreType.DMA,
        ],
    )
    def k(species_hbm, conv_hbm, out_hbm, conv_v,
          in0, in1, out0, out1, si0, si1, so0, so1):
        wid = lax.axis_index("s") * _NC + lax.axis_index("c")
        base = wid * _PER_W
        pltpu.sync_copy(conv_hbm, conv_v)

        in_bufs, out_bufs = (in0, in1), (out0, out1)
        in_sems, out_sems = (si0, si1), (so0, so1)

        def start_in(ci):
            return pltpu.async_copy(
                species_hbm.at[pl.ds(base + ci * _CHUNK, _CHUNK)],
                in_bufs[ci % 2], in_sems[ci % 2])

        def start_out(ci):
            return pltpu.async_copy(
                out_bufs[ci % 2],
                out_hbm.at[pl.ds(base + ci * _CHUNK, _CHUNK)],
                out_sems[ci % 2])

        def translate(src, dst):
            @plsc.parallel_loop(0, _CHUNK, _L, unroll=8)
            def body(i):
                idx = src[pl.ds(i, _L)]
                dst[pl.ds(i, _L)] = plsc.load_gather(conv_v, [idx])

        pending_in = {0: start_in(0)}
        pending_out = {}
        for ci in range(_NCHUNKS):
            b = ci % 2
            if ci + 1 < _NCHUNKS:
                pending_in[ci + 1] = start_in(ci + 1)
            pending_in.pop(ci).wait()
            if ci - 2 in pending_out:
                pending_out.pop(ci - 2).wait()
            translate(in_bufs[b], out_bufs[b])
            pending_out[ci] = start_out(ci)
        for ci in sorted(pending_out):
            pending_out.pop(ci).wait()

    return k(species_flat, conv)


def kernel(species, coordinates, conv_tensor):
    species_flat = species.astype(jnp.int32).reshape(-1)
    conv = jnp.pad(conv_tensor.astype(jnp.int32), (0, 128 - conv_tensor.shape[0]))
    out = _sc_table_gather(species_flat, conv)
    return out.reshape(species.shape).astype(conv_tensor.dtype), coordinates
